# Initial kernel scaffold; baseline (speedup 1.0000x reference)
import jax
import jax.numpy as jnp
from jax.experimental import pallas as pl


def kernel(node_states, a, edge_states, params, edge_index, cid):
    raise NotImplementedError("write your pallas kernel here")


# speedup vs baseline: 2.1633x; 2.1633x over previous
import functools

import jax
import jax.numpy as jnp
from jax import lax
from jax.experimental import pallas as pl
from jax.experimental.pallas import tpu as pltpu
from jax.experimental.pallas import tpu_sc as plsc

N = 10000
NE = 160000
STATE = 120
ACT = 8
H = 128
L = 4
OBSTACLE_ID = 2

BEDGE = 2000
BNODE = 2000

F32 = jnp.float32


def _dot(x, w):
    return jnp.dot(x, w, preferred_element_type=F32)


def _h0_body(ns_ref, a_ref, cid_ref, wns_ref, wna_ref, bn_ref, o_ref):
    a = jnp.where(cid_ref[:] == OBSTACLE_ID, 0.0, a_ref[:])
    o_ref[:] = _dot(ns_ref[:], wns_ref[:]) + _dot(a, wna_ref[:]) + bn_ref[:]


def _h0(ns, a, cid2, wns, wna, bn2):
    g = N // BNODE
    return pl.pallas_call(
        _h0_body,
        grid=(g,),
        in_specs=[
            pl.BlockSpec((BNODE, STATE), lambda i: (i, 0)),
            pl.BlockSpec((BNODE, ACT), lambda i: (i, 0)),
            pl.BlockSpec((BNODE, 1), lambda i: (i, 0)),
            pl.BlockSpec((STATE, H), lambda i: (0, 0)),
            pl.BlockSpec((ACT, H), lambda i: (0, 0)),
            pl.BlockSpec((1, H), lambda i: (0, 0)),
        ],
        out_specs=pl.BlockSpec((BNODE, H), lambda i: (i, 0)),
        out_shape=jax.ShapeDtypeStruct((N, H), F32),
    )(ns, a, cid2, wns, wna, bn2)


def _e0_body(es_ref, we_ref, be_ref, o_ref):
    o_ref[:] = _dot(es_ref[:], we_ref[:]) + be_ref[:]


def _e0(es, we, be2):
    g = NE // BEDGE
    return pl.pallas_call(
        _e0_body,
        grid=(g,),
        in_specs=[
            pl.BlockSpec((BEDGE, 4), lambda i: (i, 0)),
            pl.BlockSpec((4, H), lambda i: (0, 0)),
            pl.BlockSpec((1, H), lambda i: (0, 0)),
        ],
        out_specs=pl.BlockSpec((BEDGE, H), lambda i: (i, 0)),
        out_shape=jax.ShapeDtypeStruct((NE, H), F32),
    )(es, we, be2)


def _nmm_body(h_ref, wa, wb, wd, we, ba, bb, bd, be, oa, ob, od, oe):
    x = h_ref[:]
    oa[:] = _dot(x, wa[:]) + ba[:]
    ob[:] = _dot(x, wb[:]) + bb[:]
    od[:] = _dot(x, wd[:]) + bd[:]
    oe[:] = _dot(x, we[:]) + be[:]


def _nmm(h, wa, wb, wd, we, ba, bb, bd, be):
    g = N // BNODE
    wspec = pl.BlockSpec((H, H), lambda i: (0, 0))
    bspec = pl.BlockSpec((1, H), lambda i: (0, 0))
    ospec = pl.BlockSpec((BNODE, H), lambda i: (i, 0))
    oshape = jax.ShapeDtypeStruct((N, H), F32)
    return pl.pallas_call(
        _nmm_body,
        grid=(g,),
        in_specs=[pl.BlockSpec((BNODE, H), lambda i: (i, 0)),
                  wspec, wspec, wspec, wspec, bspec, bspec, bspec, bspec],
        out_specs=[ospec, ospec, ospec, ospec],
        out_shape=[oshape, oshape, oshape, oshape],
    )(h, wa, wb, wd, we, ba, bb, bd, be)


def _ep1_body(e_ref, sd_ref, se_ref, c_ref, bc_ref, epre_ref, st_ref):
    i = pl.program_id(0)
    x = _dot(e_ref[:], c_ref[:]) + bc_ref[:] + sd_ref[:] + se_ref[:]
    epre_ref[:] = x
    s = jnp.sum(x, axis=0, keepdims=True)
    q = jnp.sum(x * x, axis=0, keepdims=True)
    acc = jnp.concatenate([s, q], axis=0)

    @pl.when(i == 0)
    def _():
        st_ref[:] = acc

    @pl.when(i > 0)
    def _():
        st_ref[:] = st_ref[:] + acc


def _ep1(e, sd, se, c, bc2):
    g = NE // BEDGE
    espec = pl.BlockSpec((BEDGE, H), lambda i: (i, 0))
    return pl.pallas_call(
        _ep1_body,
        grid=(g,),
        in_specs=[espec, espec, espec,
                  pl.BlockSpec((H, H), lambda i: (0, 0)),
                  pl.BlockSpec((1, H), lambda i: (0, 0))],
        out_specs=[espec, pl.BlockSpec((2, H), lambda i: (0, 0))],
        out_shape=[jax.ShapeDtypeStruct((NE, H), F32),
                   jax.ShapeDtypeStruct((2, H), F32)],
    )(e, sd, se, c, bc2)


def _bn_scale(st_ref, g_ref, b_ref, count):
    mu = st_ref[0:1, :] / count
    var = st_ref[1:2, :] / count - mu * mu
    scale = g_ref[:] * lax.rsqrt(var + 1e-5)
    shift = b_ref[:] - mu * scale
    return scale, shift


def _ep2_body(epre_ref, sb_ref, e_ref, st_ref, ge_ref, be2_ref,
              msg_ref, sg_ref, eo_ref):
    scale, shift = _bn_scale(st_ref, ge_ref, be2_ref, float(NE))
    en = epre_ref[:] * scale + shift
    sg = jax.nn.sigmoid(en)
    sg_ref[:] = sg
    msg_ref[:] = sg * sb_ref[:]
    eo_ref[:] = e_ref[:] + jnp.maximum(en, 0.0)


def _ep2(epre, sb, e, st, ge2, bee2):
    g = NE // BEDGE
    espec = pl.BlockSpec((BEDGE, H), lambda i: (i, 0))
    vspec = pl.BlockSpec((1, H), lambda i: (0, 0))
    eshape = jax.ShapeDtypeStruct((NE, H), F32)
    return pl.pallas_call(
        _ep2_body,
        grid=(g,),
        in_specs=[espec, espec, espec,
                  pl.BlockSpec((2, H), lambda i: (0, 0)), vspec, vspec],
        out_specs=[espec, espec, espec],
        out_shape=[eshape, eshape, eshape],
    )(epre, sb, e, st, ge2, bee2)


def _ep2_last_body(epre_ref, sb_ref, st_ref, ge_ref, be2_ref,
                   msg_ref, sg_ref):
    scale, shift = _bn_scale(st_ref, ge_ref, be2_ref, float(NE))
    en = epre_ref[:] * scale + shift
    sg = jax.nn.sigmoid(en)
    sg_ref[:] = sg
    msg_ref[:] = sg * sb_ref[:]


def _ep2_last(epre, sb, st, ge2, bee2):
    g = NE // BEDGE
    espec = pl.BlockSpec((BEDGE, H), lambda i: (i, 0))
    vspec = pl.BlockSpec((1, H), lambda i: (0, 0))
    eshape = jax.ShapeDtypeStruct((NE, H), F32)
    return pl.pallas_call(
        _ep2_last_body,
        grid=(g,),
        in_specs=[espec, espec,
                  pl.BlockSpec((2, H), lambda i: (0, 0)), vspec, vspec],
        out_specs=[espec, espec],
        out_shape=[eshape, eshape],
    )(epre, sb, st, ge2, bee2)


def _npA_body(ah_ref, num_ref, den_ref, t_ref, st_ref):
    i = pl.program_id(0)
    t = ah_ref[:] + num_ref[:] / (den_ref[:] + 1e-6)
    t_ref[:] = t
    s = jnp.sum(t, axis=0, keepdims=True)
    q = jnp.sum(t * t, axis=0, keepdims=True)
    acc = jnp.concatenate([s, q], axis=0)

    @pl.when(i == 0)
    def _():
        st_ref[:] = acc

    @pl.when(i > 0)
    def _():
        st_ref[:] = st_ref[:] + acc


def _npA(ah, num, den):
    g = N // BNODE
    nspec = pl.BlockSpec((BNODE, H), lambda i: (i, 0))
    return pl.pallas_call(
        _npA_body,
        grid=(g,),
        in_specs=[nspec, nspec, nspec],
        out_specs=[nspec, pl.BlockSpec((2, H), lambda i: (0, 0))],
        out_shape=[jax.ShapeDtypeStruct((N, H), F32),
                   jax.ShapeDtypeStruct((2, H), F32)],
    )(ah, num, den)


def _npB_body(t_ref, h_ref, st_ref, gh_ref, bh_ref, ho_ref):
    scale, shift = _bn_scale(st_ref, gh_ref, bh_ref, float(N))
    hn = t_ref[:] * scale + shift
    ho_ref[:] = h_ref[:] + jnp.maximum(hn, 0.0)


def _npB(t, h, st, gh2, bh2):
    g = N // BNODE
    nspec = pl.BlockSpec((BNODE, H), lambda i: (i, 0))
    vspec = pl.BlockSpec((1, H), lambda i: (0, 0))
    return pl.pallas_call(
        _npB_body,
        grid=(g,),
        in_specs=[nspec, nspec, pl.BlockSpec((2, H), lambda i: (0, 0)),
                  vspec, vspec],
        out_specs=nspec,
        out_shape=jax.ShapeDtypeStruct((N, H), F32),
    )(t, h, st, gh2, bh2)


def _npB_last_body(t_ref, h_ref, st_ref, gh_ref, bh_ref, wout_ref, bout_ref,
                   val_ref):
    scale, shift = _bn_scale(st_ref, gh_ref, bh_ref, float(N))
    hn = t_ref[:] * scale + shift
    ho = h_ref[:] + jnp.maximum(hn, 0.0)
    val_ref[:] = _dot(ho, wout_ref[:]) + bout_ref[:]


def _npB_last(t, h, st, gh2, bh2, wout, bout2):
    g = N // BNODE
    nspec = pl.BlockSpec((BNODE, H), lambda i: (i, 0))
    vspec = pl.BlockSpec((1, H), lambda i: (0, 0))
    return pl.pallas_call(
        _npB_last_body,
        grid=(g,),
        in_specs=[nspec, nspec, pl.BlockSpec((2, H), lambda i: (0, 0)),
                  vspec, vspec,
                  pl.BlockSpec((H, 1), lambda i: (0, 0)),
                  pl.BlockSpec((1, 1), lambda i: (0, 0))],
        out_specs=pl.BlockSpec((BNODE, 1), lambda i: (i, 0)),
        out_shape=jax.ShapeDtypeStruct((N, 1), F32),
    )(t, h, st, gh2, bh2, wout, bout2)


_NW = 32
G_PER_W = NE // _NW
GCHUNK = 200
GITERS = G_PER_W // GCHUNK

_MESH = plsc.VectorSubcoreMesh(core_axis_name="c", subcore_axis_name="s")


def _gather3_body(dh, eh, bh, src, dst, sd_out, se_out, sb_out,
                  idx_s, idx_d, rows1, rows2, rows3, sem):
    wid = lax.axis_index("s") * 2 + lax.axis_index("c")
    base = wid * G_PER_W

    def step(j, carry):
        off = pl.multiple_of(base + j * GCHUNK, 8)
        pltpu.sync_copy(src.at[pl.ds(off, GCHUNK)], idx_s)
        pltpu.sync_copy(dst.at[pl.ds(off, GCHUNK)], idx_d)
        pltpu.async_copy(dh.at[idx_s], rows1, sem).wait()
        pltpu.sync_copy(rows1, sd_out.at[pl.ds(off, GCHUNK)])
        pltpu.async_copy(eh.at[idx_d], rows2, sem).wait()
        pltpu.sync_copy(rows2, se_out.at[pl.ds(off, GCHUNK)])
        pltpu.async_copy(bh.at[idx_s], rows3, sem).wait()
        pltpu.sync_copy(rows3, sb_out.at[pl.ds(off, GCHUNK)])
        return carry

    lax.fori_loop(0, GITERS, step, 0)


_gather3 = functools.partial(
    pl.kernel,
    out_type=[jax.ShapeDtypeStruct((NE, H), F32)] * 3,
    mesh=_MESH,
    scratch_types=[
        pltpu.VMEM((GCHUNK,), jnp.int32),
        pltpu.VMEM((GCHUNK,), jnp.int32),
        pltpu.VMEM((GCHUNK, H), F32),
        pltpu.VMEM((GCHUNK, H), F32),
        pltpu.VMEM((GCHUNK, H), F32),
        pltpu.SemaphoreType.DMA,
    ],
)(_gather3_body)


S_PER_T = NE // 16
SCHUNK = 400
SITERS = S_PER_T // SCHUNK
NHALF = N // 2
_GARB = NHALF
_ACC_ROWS = NHALF + 8
_CB = 200
_NB = NHALF // _CB


def _scatter_half_body(lo, msg, sg, dstidx, zrows, num_out, den_out,
                       idx_v, idx_t, rows_v, accum):
    cidx = lax.axis_index("c")
    sid = lax.axis_index("s")

    def blocks(fn):
        def bstep(m, carry):
            b = sid + m * 16

            @pl.when(b < _NB)
            def _():
                fn(pl.ds(pl.multiple_of(b * _CB, 8), _CB))

            return carry

        lax.fori_loop(0, (_NB + 15) // 16, bstep, 0)

    blocks(lambda sl: pltpu.sync_copy(zrows, accum.at[sl]))
    plsc.subcore_barrier()

    def sweep(src_ref, out_ref):
        def step(j, carry):
            off = pl.multiple_of(sid * S_PER_T + j * SCHUNK, 8)
            pltpu.sync_copy(dstidx.at[pl.ds(off, SCHUNK)], idx_v)
            pltpu.sync_copy(src_ref.at[pl.ds(off, SCHUNK)], rows_v)

            def rebase(k, carry2):
                sl = pl.ds(pl.multiple_of(k * 16, 8), 16)
                v = idx_v[sl]
                inr = (v >= lo) & (v < lo + NHALF)
                idx_t[sl] = jnp.where(inr, v - lo, _GARB)
                return carry2

            lax.fori_loop(0, SCHUNK // 16, rebase, 0)
            pltpu.sync_copy(rows_v, accum.at[idx_t], add=True)
            return carry

        lax.fori_loop(0, SITERS, step, 0)
        plsc.subcore_barrier()
        blocks(lambda sl: pltpu.sync_copy(accum.at[sl], out_ref.at[sl]))

    @pl.when(cidx == 0)
    def _():
        sweep(msg, num_out)

    @pl.when(cidx == 1)
    def _():
        sweep(sg, den_out)


def _make_scatter_half(lo):
    return functools.partial(
        pl.kernel,
        out_type=[jax.ShapeDtypeStruct((NHALF, H), F32)] * 2,
        mesh=_MESH,
        scratch_types=[
            pltpu.VMEM((SCHUNK,), jnp.int32),
            pltpu.VMEM((SCHUNK,), jnp.int32),
            pltpu.VMEM((SCHUNK, H), F32),
            pltpu.VMEM_SHARED((_ACC_ROWS, H), F32),
        ],
    )(functools.partial(_scatter_half_body, lo))


_scatter_lo = _make_scatter_half(0)
_scatter_hi = _make_scatter_half(NHALF)


def kernel(node_states, a, edge_states, params, edge_index, cid):
    p = params
    src = edge_index[0]
    dst = edge_index[1]
    cid2 = cid.reshape(N, 1)

    def row(v):
        return v.reshape(1, -1)

    h = _h0(node_states, a, cid2, p["Wn"][:STATE], p["Wn"][STATE:],
            row(p["bn"]))
    e = _e0(edge_states, p["We"], row(p["be"]))
    zrows = jnp.zeros((_CB, H), F32)

    out = None
    for l in range(L):
        ah, bh_, dh, eh = _nmm(
            h, p["A"][l], p["B"][l], p["D"][l], p["Ex"][l],
            row(p["bA"][l]), row(p["bB"][l]), row(p["bD"][l]),
            row(p["bEx"][l]))
        sd, se, sb = _gather3(dh, eh, bh_, src, dst)
        epre, st_e = _ep1(e, sd, se, p["C"][l], row(p["bC"][l]))
        if l == L - 1:
            msg, sg = _ep2_last(epre, sb, st_e, row(p["ge"][l]),
                                row(p["be2"][l]))
        else:
            msg, sg, e = _ep2(epre, sb, e, st_e, row(p["ge"][l]),
                              row(p["be2"][l]))
        num_lo, den_lo = _scatter_lo(msg, sg, dst, zrows)
        num_hi, den_hi = _scatter_hi(msg, sg, dst, zrows)
        num = jnp.concatenate([num_lo, num_hi], axis=0)
        den = jnp.concatenate([den_lo, den_hi], axis=0)
        t, st_n = _npA(ah, num, den)
        if l == L - 1:
            out = _npB_last(t, h, st_n, row(p["gh"][l]), row(p["bh"][l]),
                            p["Wout"], p["bout"].reshape(1, 1))
        else:
            h = _npB(t, h, st_n, row(p["gh"][l]), row(p["bh"][l]))
    return out
